# Initial kernel scaffold; baseline (speedup 1.0000x reference)
import jax
import jax.numpy as jnp
from jax.experimental import pallas as pl


def kernel(x, batch, W):
    raise NotImplementedError("write your pallas kernel here")


# speedup vs baseline: 1.8832x; 1.8832x over previous
import functools

import jax
import jax.numpy as jnp
from jax import lax
from jax.experimental import pallas as pl
from jax.experimental.pallas import tpu as pltpu
from jax.experimental.pallas import tpu_sc as plsc

_G = 512
_LANES = 16
_NC = 2
_NS = 16


def _segment_max_sc(x, batch):
    n, d = x.shape
    nw = _NC * _NS
    rows_per_w = n // nw
    blk = 125
    nblk = rows_per_w // blk
    acc_words = _G * d
    ids_pad = rows_per_w + 8 - rows_per_w % 8

    batch2 = jnp.pad(batch.reshape(nw, rows_per_w),
                     ((0, 0), (0, ids_pad - rows_per_w)))

    mesh = plsc.VectorSubcoreMesh(
        core_axis_name="c", subcore_axis_name="s",
        num_cores=_NC, num_subcores=_NS)

    @functools.partial(
        pl.kernel,
        mesh=mesh,
        compiler_params=pltpu.CompilerParams(use_tc_tiling_on_sc=False),
        out_type=jax.ShapeDtypeStruct((nw, acc_words), jnp.float32),
        scratch_types=[
            pltpu.VMEM((acc_words,), jnp.float32),
            pltpu.VMEM((ids_pad + _LANES,), jnp.int32),
            pltpu.VMEM((2, blk, d), jnp.float32),
            pltpu.SemaphoreType.DMA,
            pltpu.SemaphoreType.DMA,
        ],
    )
    def seg_max(x_hbm, b_hbm, out_hbm, acc, bids, xbuf, sem0, sem1):
        wid = lax.axis_index("s") * _NC + lax.axis_index("c")
        base = wid * rows_per_w

        pltpu.sync_copy(b_hbm.at[wid], bids.at[pl.ds(0, ids_pad)])

        neg = jnp.full((_LANES,), -jnp.inf, dtype=jnp.float32)

        def init_body(i, _):
            acc[pl.ds(i * _LANES, _LANES)] = neg
            return 0

        lax.fori_loop(0, acc_words // _LANES, init_body, 0)

        sems = (sem0, sem1)
        copies = [None, None]
        copies[0] = pltpu.async_copy(
            x_hbm.at[pl.ds(base, blk)], xbuf.at[0], sem0)
        for bi in range(nblk):
            slot = bi % 2
            if bi + 1 < nblk:
                nslot = (bi + 1) % 2
                copies[nslot] = pltpu.async_copy(
                    x_hbm.at[pl.ds(base + (bi + 1) * blk, blk)],
                    xbuf.at[nslot], sems[nslot])
            copies[slot].wait()

            def row_body(r, _, bi=bi, slot=slot):
                seg = bids[pl.ds(bi * blk + r, _LANES)][0]
                off = seg * d
                for j in range(d // _LANES):
                    v = xbuf[slot, r, pl.ds(j * _LANES, _LANES)]
                    a = acc[pl.ds(off + j * _LANES, _LANES)]
                    acc[pl.ds(off + j * _LANES, _LANES)] = jnp.maximum(a, v)
                return 0

            lax.fori_loop(0, blk, row_body, 0)

        pltpu.sync_copy(acc, out_hbm.at[wid])

    return seg_max(x, batch2)


def _combine_and_matmul_tc(accs, w_pad):
    def body(a_ref, w_ref, o_ref):
        hg = jnp.max(a_ref[...], axis=0)
        o_ref[...] = jnp.dot(hg, w_ref[...],
                             preferred_element_type=jnp.float32)

    return pl.pallas_call(
        body,
        out_shape=jax.ShapeDtypeStruct((_G, 128), jnp.float32),
    )(accs, w_pad)


def kernel(x, batch, W):
    n, d = x.shape
    n_classes = W.shape[0]
    batch = batch.astype(jnp.int32)
    accs = _segment_max_sc(x, batch).reshape(_NC * _NS, _G, d)
    w_pad = jnp.zeros((d, 128), jnp.float32).at[:, :n_classes].set(W.T)
    logits = _combine_and_matmul_tc(accs, w_pad)
    return logits[:, :n_classes]


# speedup vs baseline: 2.0358x; 1.0811x over previous
import functools

import jax
import jax.numpy as jnp
from jax import lax
from jax.experimental import pallas as pl
from jax.experimental.pallas import tpu as pltpu
from jax.experimental.pallas import tpu_sc as plsc

_G = 512
_LANES = 16
_NC = 2
_NS = 16


def _segment_max_sc(x, batch):
    n, d = x.shape
    nw = _NC * _NS
    rows_per_w = n // nw
    blk = 125
    nblk = rows_per_w // blk
    acc_words = _G * d
    ids_pad = rows_per_w + 8 - rows_per_w % 8

    batch2 = jnp.pad(batch.reshape(nw, rows_per_w),
                     ((0, 0), (0, ids_pad - rows_per_w)))

    mesh = plsc.VectorSubcoreMesh(
        core_axis_name="c", subcore_axis_name="s",
        num_cores=_NC, num_subcores=_NS)

    neg_init = jnp.full((acc_words,), -jnp.inf, jnp.float32)

    @functools.partial(
        pl.kernel,
        mesh=mesh,
        compiler_params=pltpu.CompilerParams(use_tc_tiling_on_sc=False),
        out_type=jax.ShapeDtypeStruct((nw, acc_words), jnp.float32),
        scratch_types=[
            pltpu.VMEM((acc_words,), jnp.float32),
            pltpu.VMEM((ids_pad + _LANES,), jnp.int32),
            pltpu.VMEM((2, blk, d), jnp.float32),
            pltpu.SemaphoreType.DMA,
            pltpu.SemaphoreType.DMA,
            pltpu.SemaphoreType.DMA,
        ],
    )
    def seg_max(x_hbm, b_hbm, neg_hbm, out_hbm, acc, bids, xbuf,
                sem0, sem1, sem2):
        wid = lax.axis_index("s") * _NC + lax.axis_index("c")
        base = wid * rows_per_w

        init_cp = pltpu.async_copy(neg_hbm, acc, sem2)
        pltpu.sync_copy(b_hbm.at[wid], bids.at[pl.ds(0, ids_pad)])

        sems = (sem0, sem1)
        copies = [None, None]
        copies[0] = pltpu.async_copy(
            x_hbm.at[pl.ds(base, blk)], xbuf.at[0], sem0)
        init_cp.wait()

        nvec = d // _LANES
        carry = (jnp.int32(-1),) + tuple(
            jnp.full((_LANES,), -jnp.inf, jnp.float32) for _ in range(nvec))
        for bi in range(nblk):
            slot = bi % 2
            if bi + 1 < nblk:
                nslot = (bi + 1) % 2
                copies[nslot] = pltpu.async_copy(
                    x_hbm.at[pl.ds(base + (bi + 1) * blk, blk)],
                    xbuf.at[nslot], sems[nslot])
            copies[slot].wait()

            def row_body(r, c, bi=bi, slot=slot):
                cur, m = c[0], c[1:]
                seg = bids[pl.ds(bi * blk + r, _LANES)][0]
                eq = seg == cur
                off = seg * d
                new = []
                for j in range(nvec):
                    xv = xbuf[slot, r, pl.ds(j * _LANES, _LANES)]
                    mv = jnp.where(eq, jnp.maximum(m[j], xv), xv)
                    acc[pl.ds(off + j * _LANES, _LANES)] = mv
                    new.append(mv)
                return (seg, *new)

            carry = lax.fori_loop(0, blk, row_body, carry)

        pltpu.sync_copy(acc, out_hbm.at[wid])

    return seg_max(x, batch2, neg_init)


def _combine_and_matmul_tc(accs, w_pad):
    def body(a_ref, w_ref, o_ref):
        hg = jnp.max(a_ref[...], axis=0)
        o_ref[...] = jnp.dot(hg, w_ref[...],
                             preferred_element_type=jnp.float32)

    return pl.pallas_call(
        body,
        out_shape=jax.ShapeDtypeStruct((_G, 128), jnp.float32),
    )(accs, w_pad)


def kernel(x, batch, W):
    n, d = x.shape
    n_classes = W.shape[0]
    batch = batch.astype(jnp.int32)
    accs = _segment_max_sc(x, batch).reshape(_NC * _NS, _G, d)
    w_pad = jnp.zeros((d, 128), jnp.float32).at[:, :n_classes].set(W.T)
    logits = _combine_and_matmul_tc(accs, w_pad)
    return logits[:, :n_classes]


# speedup vs baseline: 2.4255x; 1.1914x over previous
import functools

import jax
import jax.numpy as jnp
from jax import lax
from jax.experimental import pallas as pl
from jax.experimental.pallas import tpu as pltpu
from jax.experimental.pallas import tpu_sc as plsc

_G = 512
_LANES = 16
_NC = 2
_NS = 16


def _segment_max_sc(x, batch):
    n, d = x.shape
    nw = _NC * _NS
    rows_per_w = n // nw
    blk = 125
    nblk = rows_per_w // blk
    acc_words = _G * d
    ids_pad = rows_per_w + 8 - rows_per_w % 8

    batch2 = jnp.pad(batch.reshape(nw, rows_per_w),
                     ((0, 0), (0, ids_pad - rows_per_w)))

    mesh = plsc.VectorSubcoreMesh(
        core_axis_name="c", subcore_axis_name="s",
        num_cores=_NC, num_subcores=_NS)

    neg_init = jnp.full((acc_words,), -jnp.inf, jnp.float32)

    @functools.partial(
        pl.kernel,
        mesh=mesh,
        compiler_params=pltpu.CompilerParams(use_tc_tiling_on_sc=False),
        out_type=jax.ShapeDtypeStruct((nw, acc_words), jnp.float32),
        scratch_types=[
            pltpu.VMEM((acc_words,), jnp.float32),
            pltpu.VMEM((ids_pad + _LANES,), jnp.int32),
            pltpu.VMEM((2, blk, d), jnp.float32),
            pltpu.SemaphoreType.DMA,
            pltpu.SemaphoreType.DMA,
            pltpu.SemaphoreType.DMA,
        ],
    )
    def seg_max(x_hbm, b_hbm, neg_hbm, out_hbm, acc, bids, xbuf,
                sem0, sem1, sem2):
        wid = lax.axis_index("s") * _NC + lax.axis_index("c")
        base = wid * rows_per_w

        init_cp = pltpu.async_copy(neg_hbm, acc, sem2)
        pltpu.sync_copy(b_hbm.at[wid], bids.at[pl.ds(0, ids_pad)])

        pltpu.async_copy(x_hbm.at[pl.ds(base, blk)], xbuf.at[0], sem0)
        pltpu.async_copy(x_hbm.at[pl.ds(base + blk, blk)], xbuf.at[1], sem1)
        init_cp.wait()

        nvec = d // _LANES
        ngrp = blk // _LANES
        tail = blk - ngrp * _LANES

        def rows_16(row0, slot, idsv, nrows, cur, m):
            for k in range(nrows):
                seg = idsv[k]
                eq = seg == cur
                off = seg * d
                mn = []
                for j in range(nvec):
                    xv = xbuf[slot, row0 + k, pl.ds(j * _LANES, _LANES)]
                    mv = jnp.where(eq, jnp.maximum(m[j], xv), xv)
                    acc[pl.ds(off + j * _LANES, _LANES)] = mv
                    mn.append(mv)
                m = mn
                cur = seg
            return cur, m

        def block_body(bi, c):
            cur, m = c[0], list(c[1:])
            slot = lax.rem(bi, 2)
            @pl.when(slot == 0)
            def _():
                pltpu.make_async_copy(
                    x_hbm.at[pl.ds(base, blk)], xbuf.at[0], sem0).wait()

            @pl.when(slot == 1)
            def _():
                pltpu.make_async_copy(
                    x_hbm.at[pl.ds(base, blk)], xbuf.at[1], sem1).wait()

            @pl.when((slot == 0) & (bi + 2 < nblk))
            def _():
                pltpu.async_copy(
                    x_hbm.at[pl.ds(base + (bi + 2) * blk, blk)],
                    xbuf.at[0], sem0)

            @pl.when((slot == 1) & (bi + 2 < nblk))
            def _():
                pltpu.async_copy(
                    x_hbm.at[pl.ds(base + (bi + 2) * blk, blk)],
                    xbuf.at[1], sem1)

            def group_body(g, cg):
                cur, m = cg[0], list(cg[1:])
                idsv = bids[pl.ds(bi * blk + g * _LANES, _LANES)]
                cur, m = rows_16(g * _LANES, slot, idsv, _LANES, cur, m)
                return (cur, *m)

            cg = lax.fori_loop(0, ngrp, group_body, (cur, *m))
            cur, m = cg[0], list(cg[1:])
            idsv = bids[pl.ds(bi * blk + ngrp * _LANES, _LANES)]
            cur, m = rows_16(ngrp * _LANES, slot, idsv, tail, cur, m)
            return (cur, *m)

        carry0 = (jnp.int32(-1),) + tuple(
            jnp.full((_LANES,), -jnp.inf, jnp.float32) for _ in range(nvec))
        lax.fori_loop(0, nblk, block_body, carry0)

        pltpu.sync_copy(acc, out_hbm.at[wid])

    return seg_max(x, batch2, neg_init)


def _combine_and_matmul_tc(accs, w_pad):
    def body(a_ref, w_ref, o_ref):
        hg = jnp.max(a_ref[...], axis=0)
        o_ref[...] = jnp.dot(hg, w_ref[...],
                             preferred_element_type=jnp.float32)

    return pl.pallas_call(
        body,
        out_shape=jax.ShapeDtypeStruct((_G, 128), jnp.float32),
    )(accs, w_pad)


def kernel(x, batch, W):
    n, d = x.shape
    n_classes = W.shape[0]
    batch = batch.astype(jnp.int32)
    accs = _segment_max_sc(x, batch).reshape(_NC * _NS, _G, d)
    w_pad = jnp.zeros((d, 128), jnp.float32).at[:, :n_classes].set(W.T)
    logits = _combine_and_matmul_tc(accs, w_pad)
    return logits[:, :n_classes]
